# Initial kernel scaffold; baseline (speedup 1.0000x reference)
import jax
import jax.numpy as jnp
from jax.experimental import pallas as pl


def kernel(token_level_features, W, b, chunk_lens):
    raise NotImplementedError("write your pallas kernel here")


# speedup vs baseline: 6.2628x; 6.2628x over previous
import jax
import jax.numpy as jnp
from jax.experimental import pallas as pl

_B, _S, _D = 16, 2048, 128
_C, _L = 256, 8
_EXT = 512


def _conv_body(ext_ref, x_ref, wt_ref, bias_ref, out_ref):
    x = x_ref[0]
    ext = ext_ref[0]
    s2 = jax.lax.broadcasted_iota(jnp.int32, (_S, _EXT), 0)
    e = jnp.broadcast_to(ext, (_S, _EXT))
    is_start = jnp.max((e == s2).astype(jnp.float32), axis=1, keepdims=True)
    is_start_n = jnp.max((e == s2 + 1).astype(jnp.float32), axis=1, keepdims=True)
    total = jnp.max(ext)
    sv = jax.lax.broadcasted_iota(jnp.int32, (_S, 1), 0)
    valid = (sv < total).astype(jnp.float32)
    m_l = 1.0 - is_start
    m_r = 1.0 - is_start_n

    xl = jnp.concatenate([x[:1], x[:-1]], axis=0)
    xr = jnp.concatenate([x[1:], x[-1:]], axis=0)

    acc = jnp.dot(xl * m_l, wt_ref[0], preferred_element_type=jnp.float32)
    acc = acc + jnp.dot(x, wt_ref[1], preferred_element_type=jnp.float32)
    acc = acc + jnp.dot(xr * m_r, wt_ref[2], preferred_element_type=jnp.float32)
    acc = acc + bias_ref[0][None, :]
    out_ref[0] = jnp.maximum(acc, 0.0) * valid


def kernel(token_level_features, W, b, chunk_lens):
    x = token_level_features
    cl = chunk_lens.astype(jnp.int32)
    csum = jnp.cumsum(cl, axis=1)
    total = csum[:, -1:]
    offsets = jnp.concatenate(
        [jnp.zeros((_B, 1), jnp.int32), csum[:, :-1]], axis=1)
    ext = jnp.concatenate(
        [offsets, jnp.broadcast_to(total, (_B, _EXT - _C))], axis=1)
    ext3 = ext.reshape(_B, 1, _EXT)
    wt = jnp.transpose(W, (2, 1, 0))
    bias2 = b.reshape(1, _D)

    out = pl.pallas_call(
        _conv_body,
        grid=(_B,),
        in_specs=[
            pl.BlockSpec((1, 1, _EXT), lambda i: (i, 0, 0)),
            pl.BlockSpec((1, _S, _D), lambda i: (i, 0, 0)),
            pl.BlockSpec((3, _D, _D), lambda i: (0, 0, 0)),
            pl.BlockSpec((1, _D), lambda i: (0, 0)),
        ],
        out_specs=pl.BlockSpec((1, _S, _D), lambda i: (i, 0, 0)),
        out_shape=jax.ShapeDtypeStruct((_B, _S, _D), x.dtype),
    )(ext3, x, wt, bias2)
    return out


# speedup vs baseline: 7.4433x; 1.1885x over previous
import jax
import jax.numpy as jnp
from jax.experimental import pallas as pl

_B, _S, _D = 16, 2048, 128
_C, _L = 256, 8
_EXT = 384


def _conv_body(ext_ref, x_ref, wt_ref, bias_ref, out_ref):
    x = x_ref[0]
    ext = ext_ref[0]
    s2 = jax.lax.broadcasted_iota(jnp.int32, (_S, _EXT), 0)
    e = jnp.broadcast_to(ext, (_S, _EXT))
    m = 1.0 - jnp.max((e == s2).astype(jnp.float32), axis=1, keepdims=True)
    total = jnp.max(ext)

    xl = jnp.concatenate([x[:1], x[:-1]], axis=0) * m
    xm = x * m
    xr = jnp.concatenate([xm[1:], xm[-1:]], axis=0)

    acc = jnp.dot(xl, wt_ref[0], preferred_element_type=jnp.float32)
    acc = acc + jnp.dot(x, wt_ref[1], preferred_element_type=jnp.float32)
    acc = acc + jnp.dot(xr, wt_ref[2], preferred_element_type=jnp.float32)
    acc = acc + bias_ref[0][None, :]
    sv = jax.lax.broadcasted_iota(jnp.int32, (_S, _D), 0)
    out_ref[0] = jnp.where(sv < total, jnp.maximum(acc, 0.0), 0.0)


def kernel(token_level_features, W, b, chunk_lens):
    x = token_level_features
    cl = chunk_lens.astype(jnp.int32)
    csum = jnp.cumsum(cl, axis=1)
    total = csum[:, -1:]
    offsets = jnp.concatenate(
        [jnp.zeros((_B, 1), jnp.int32), csum[:, :-1]], axis=1)
    ext = jnp.concatenate(
        [offsets, jnp.broadcast_to(total, (_B, _EXT - _C))], axis=1)
    ext3 = ext.reshape(_B, 1, _EXT)
    wt = jnp.transpose(W, (2, 1, 0))
    bias2 = b.reshape(1, _D)

    out = pl.pallas_call(
        _conv_body,
        grid=(_B,),
        in_specs=[
            pl.BlockSpec((1, 1, _EXT), lambda i: (i, 0, 0)),
            pl.BlockSpec((1, _S, _D), lambda i: (i, 0, 0)),
            pl.BlockSpec((3, _D, _D), lambda i: (0, 0, 0)),
            pl.BlockSpec((1, _D), lambda i: (0, 0)),
        ],
        out_specs=pl.BlockSpec((1, _S, _D), lambda i: (i, 0, 0)),
        out_shape=jax.ShapeDtypeStruct((_B, _S, _D), x.dtype),
    )(ext3, x, wt, bias2)
    return out
